# baseline (speedup 1.0000x reference)
import functools

import jax
import jax.numpy as jnp
from jax import lax
from jax.experimental import pallas as pl
from jax.experimental.pallas import tpu as pltpu
from jax.experimental.pallas import tpu_sc as plsc

F32 = jnp.float32
_NC, _NS = 2, 16
_NW = _NC * _NS
_CHUNK = 80


def _fold_mlp(layers, eps=1e-5):
    folded = []
    n = len(layers)
    for i, p in enumerate(layers):
        W, b = p["W"], p["b"]
        if i < n - 1:
            s = p["g"] * lax.rsqrt(p["v"] + eps)
            W = W * s[None, :]
            b = b * s + (p["be"] - p["m"] * s)
        folded.append((W, b.reshape(1, -1)))
    return folded


def _sc_gather2(x, row, col):
    n, d = x.shape
    e = row.shape[0]
    epw = e // _NW
    c = _CHUNK
    mesh = plsc.VectorSubcoreMesh(core_axis_name="c", subcore_axis_name="s")

    @functools.partial(
        pl.kernel,
        mesh=mesh,
        out_type=(jax.ShapeDtypeStruct((e, d), F32),
                  jax.ShapeDtypeStruct((e, d), F32)),
        scratch_types=[
            pltpu.VMEM((c,), jnp.int32), pltpu.VMEM((c, d), F32),
            pltpu.VMEM((c,), jnp.int32), pltpu.VMEM((c, d), F32),
            pltpu.SemaphoreType.DMA, pltpu.SemaphoreType.DMA,
        ],
    )
    def k(x_hbm, row_hbm, col_hbm, outr_hbm, outc_hbm,
          idx_r, buf_r, idx_c, buf_c, sem_r, sem_c):
        wid = lax.axis_index("s") * _NC + lax.axis_index("c")
        base = wid * epw

        @pl.loop(0, epw, step=c)
        def _(off):
            b = base + off
            pltpu.sync_copy(row_hbm.at[pl.ds(b, c)], idx_r)
            pltpu.sync_copy(col_hbm.at[pl.ds(b, c)], idx_c)
            cp_r = pltpu.async_copy(x_hbm.at[idx_r], buf_r, sem_r)
            cp_c = pltpu.async_copy(x_hbm.at[idx_c], buf_c, sem_c)
            cp_r.wait()
            cp_c.wait()
            pltpu.sync_copy(buf_r, outr_hbm.at[pl.ds(b, c)])
            pltpu.sync_copy(buf_c, outc_hbm.at[pl.ds(b, c)])

    return k(x, row, col)


def _sc_segment_sum(msg, col, n):
    e, d = msg.shape
    epw = e // _NW
    c = _CHUNK
    npad = -(-n // 128) * 128
    nps = npad // _NS
    zeros = jnp.zeros((npad, d), F32)
    mesh = plsc.VectorSubcoreMesh(core_axis_name="c", subcore_axis_name="s")

    @functools.partial(
        pl.kernel,
        mesh=mesh,
        out_type=jax.ShapeDtypeStruct((_NC * npad, d), F32),
        scratch_types=[
            pltpu.VMEM((c,), jnp.int32), pltpu.VMEM((c, d), F32),
            pltpu.VMEM_SHARED((npad, d), F32),
            pltpu.SemaphoreType.DMA,
        ],
        compiler_params=pltpu.CompilerParams(use_tc_tiling_on_sc=False),
    )
    def k(msg_hbm, col_hbm, z_hbm, out_hbm, idx_v, buf_v, acc_sh, sem):
        cid = lax.axis_index("c")
        sid = lax.axis_index("s")
        wid = sid * _NC + cid
        rbase = sid * nps
        pltpu.sync_copy(z_hbm.at[pl.ds(rbase, nps)], acc_sh.at[pl.ds(rbase, nps)])
        plsc.subcore_barrier()

        base = wid * epw

        @pl.loop(0, epw, step=c)
        def _(off):
            b = base + off
            pltpu.sync_copy(col_hbm.at[pl.ds(b, c)], idx_v)
            pltpu.sync_copy(msg_hbm.at[pl.ds(b, c)], buf_v)
            pltpu.sync_copy(buf_v, acc_sh.at[idx_v], add=True)

        plsc.subcore_barrier()
        pltpu.sync_copy(acc_sh.at[pl.ds(rbase, nps)],
                        out_hbm.at[pl.ds(cid * npad + rbase, nps)])

    out = k(msg, col, zeros)
    return out[:n], out[npad:npad + n]


def _tc_edge(xr, xc, ea, ew, uw, block_e=3200):
    e, d = xr.shape
    de = ea.shape[1]
    A1, B1, C1 = ew[0][0][:d], ew[0][0][d:2 * d], ew[0][0][2 * d:]
    b1 = ew[0][1]
    W2, b2 = ew[1]
    W3, b3 = ew[2]
    U1a, U1b = uw[0][0][:de], uw[0][0][de:]
    ub1 = uw[0][1]
    U2, ub2 = uw[1]
    U3, ub3 = uw[2]
    dh = W3.shape[1]

    def body(xr_ref, xc_ref, ea_ref, A1r, B1r, C1r, b1r, W2r, b2r, W3r, b3r,
             U1ar, U1br, ub1r, U2r, ub2r, U3r, ub3r, msg_ref, ea2_ref):
        xr_ = xr_ref[...]
        xc_ = xc_ref[...]
        ea_ = ea_ref[...]
        dot = lambda a, b: jnp.dot(a, b, preferred_element_type=F32,
                                   precision=lax.Precision.HIGHEST)
        xrA = dot(xr_, A1r[...])
        xrB = dot(xr_, B1r[...])
        xcA = dot(xc_, A1r[...])
        xcB = dot(xc_, B1r[...])
        h = jnp.maximum(xrA + xcB + dot(ea_, C1r[...]) + b1r[...], 0.0)
        h = jnp.maximum(dot(h, W2r[...]) + b2r[...], 0.0)
        m1 = dot(h, W3r[...]) + b3r[...]
        t = jnp.maximum(dot(ea_, U1ar[...]) + dot(m1, U1br[...]) + ub1r[...], 0.0)
        t = jnp.maximum(dot(t, U2r[...]) + ub2r[...], 0.0)
        ea2 = dot(t, U3r[...]) + ub3r[...]
        ea2_ref[...] = ea2
        g = jnp.maximum(xcA + xrB + dot(ea2, C1r[...]) + b1r[...], 0.0)
        g = jnp.maximum(dot(g, W2r[...]) + b2r[...], 0.0)
        msg_ref[...] = dot(g, W3r[...]) + b3r[...]

    full = lambda a: pl.BlockSpec(a.shape, lambda i: (0,) * a.ndim)
    weights = [A1, B1, C1, b1, W2, b2, W3, b3, U1a, U1b, ub1, U2, ub2, U3, ub3]
    grid = (e // block_e,)
    return pl.pallas_call(
        body,
        grid=grid,
        in_specs=[
            pl.BlockSpec((block_e, d), lambda i: (i, 0)),
            pl.BlockSpec((block_e, d), lambda i: (i, 0)),
            pl.BlockSpec((block_e, de), lambda i: (i, 0)),
        ] + [full(w) for w in weights],
        out_specs=[
            pl.BlockSpec((block_e, dh), lambda i: (i, 0)),
            pl.BlockSpec((block_e, de), lambda i: (i, 0)),
        ],
        out_shape=[
            jax.ShapeDtypeStruct((e, dh), F32),
            jax.ShapeDtypeStruct((e, de), F32),
        ],
    )(xr, xc, ea, *weights)


def _tc_node(x, p0, p1, nw, residual):
    n, d = x.shape
    N1a, N1b = nw[0][0][:d], nw[0][0][d:]
    nb1 = nw[0][1]
    N2, nb2 = nw[1]
    N3, nb3 = nw[2]

    def body(x_ref, p0_ref, p1_ref, N1ar, N1br, nb1r, N2r, nb2r, N3r, nb3r,
             out_ref):
        dot = lambda a, b: jnp.dot(a, b, preferred_element_type=F32,
                                   precision=lax.Precision.HIGHEST)
        agg = p0_ref[...] + p1_ref[...]
        xv = x_ref[...]
        h = jnp.maximum(dot(xv, N1ar[...]) + dot(agg, N1br[...]) + nb1r[...], 0.0)
        h = jnp.maximum(dot(h, N2r[...]) + nb2r[...], 0.0)
        o = dot(h, N3r[...]) + nb3r[...]
        out_ref[...] = o + xv if residual else o

    return pl.pallas_call(
        body,
        out_shape=jax.ShapeDtypeStruct((n, d), F32),
    )(x, p0, p1, N1a, N1b, nb1, N2, nb2, N3, nb3)


def kernel(x, edge_attr, edge_index, params):
    row = edge_index[0]
    col = edge_index[1]
    n = x.shape[0]
    for i, lp in enumerate(params):
        ew = _fold_mlp(lp["edge_mlp"])
        uw = _fold_mlp(lp["edge_update_mlp"])
        nw = _fold_mlp(lp["node_mlp"])
        xr, xc = _sc_gather2(x, row, col)
        msg, edge_attr = _tc_edge(xr, xc, edge_attr, ew, uw)
        p0, p1 = _sc_segment_sum(msg, col, n)
        x = _tc_node(x, p0, p1, nw, residual=(i > 0))
    return x, edge_attr


# speedup vs baseline: 1.9479x; 1.9479x over previous
import functools

import jax
import jax.numpy as jnp
from jax import lax
from jax.experimental import pallas as pl
from jax.experimental.pallas import tpu as pltpu
from jax.experimental.pallas import tpu_sc as plsc

F32 = jnp.float32
_NC, _NS = 2, 16
_NW = _NC * _NS
_CHUNK = 80
_HI = lax.Precision.HIGHEST
_MED = lax.Precision.HIGH


def _fold_mlp(layers, eps=1e-5):
    folded = []
    n = len(layers)
    for i, p in enumerate(layers):
        W, b = p["W"], p["b"]
        if i < n - 1:
            s = p["g"] * lax.rsqrt(p["v"] + eps)
            W = W * s[None, :]
            b = b * s + (p["be"] - p["m"] * s)
        folded.append((W, b.reshape(1, -1)))
    return folded


def _prep_layer(lp):
    ew = _fold_mlp(lp["edge_mlp"])
    uw = _fold_mlp(lp["edge_update_mlp"])
    nw = _fold_mlp(lp["node_mlp"])
    W1, b1 = ew[0]
    d = 128
    de = 16
    A1, B1, C1 = W1[:d], W1[d:2 * d], W1[2 * d:]
    W2, b2 = ew[1]
    W3, b3 = ew[2]
    U1, ub1 = uw[0]
    U1a, U1b = U1[:de], U1[de:]
    V = jnp.concatenate([U1a, W3 @ U1b], axis=0)
    vb = ub1 + b3 @ U1b
    U2, ub2 = uw[1]
    U3, ub3 = uw[2]
    Pw = jnp.concatenate([A1, B1], axis=1)
    return {
        "Pw": Pw, "C1": C1, "b1": b1, "W2": W2, "b2": b2, "W3": W3, "b3": b3,
        "V": V, "vb": vb, "U2": U2, "ub2": ub2, "U3": U3, "ub3": ub3,
        "nw": nw,
    }


def _sc_gather2(p, row, col):
    n, d = p.shape
    e = row.shape[0]
    epw = e // _NW
    c = _CHUNK
    mesh = plsc.VectorSubcoreMesh(core_axis_name="c", subcore_axis_name="s")

    @functools.partial(
        pl.kernel,
        mesh=mesh,
        out_type=(jax.ShapeDtypeStruct((e, d), F32),
                  jax.ShapeDtypeStruct((e, d), F32)),
        scratch_types=[
            pltpu.VMEM((c,), jnp.int32), pltpu.VMEM((c, d), F32),
            pltpu.VMEM((c,), jnp.int32), pltpu.VMEM((c, d), F32),
            pltpu.SemaphoreType.DMA, pltpu.SemaphoreType.DMA,
        ],
    )
    def k(p_hbm, row_hbm, col_hbm, outr_hbm, outc_hbm,
          idx_r, buf_r, idx_c, buf_c, sem_r, sem_c):
        wid = lax.axis_index("s") * _NC + lax.axis_index("c")
        base = wid * epw

        @pl.loop(0, epw, step=c)
        def _(off):
            b = base + off
            pltpu.sync_copy(row_hbm.at[pl.ds(b, c)], idx_r)
            pltpu.sync_copy(col_hbm.at[pl.ds(b, c)], idx_c)
            cp_r = pltpu.async_copy(p_hbm.at[idx_r], buf_r, sem_r)
            cp_c = pltpu.async_copy(p_hbm.at[idx_c], buf_c, sem_c)
            cp_r.wait()
            cp_c.wait()
            pltpu.sync_copy(buf_r, outr_hbm.at[pl.ds(b, c)])
            pltpu.sync_copy(buf_c, outc_hbm.at[pl.ds(b, c)])

    return k(p, row, col)


def _sc_segment_sum(msg, col, n):
    e, d = msg.shape
    epw = e // _NW
    c = _CHUNK
    npad = -(-n // 128) * 128
    nps = npad // _NS
    zeros = jnp.zeros((npad, d), F32)
    mesh = plsc.VectorSubcoreMesh(core_axis_name="c", subcore_axis_name="s")

    @functools.partial(
        pl.kernel,
        mesh=mesh,
        out_type=jax.ShapeDtypeStruct((_NC * npad, d), F32),
        scratch_types=[
            pltpu.VMEM((c,), jnp.int32), pltpu.VMEM((c, d), F32),
            pltpu.VMEM_SHARED((npad, d), F32),
            pltpu.SemaphoreType.DMA,
        ],
        compiler_params=pltpu.CompilerParams(use_tc_tiling_on_sc=False),
    )
    def k(msg_hbm, col_hbm, z_hbm, out_hbm, idx_v, buf_v, acc_sh, sem):
        cid = lax.axis_index("c")
        sid = lax.axis_index("s")
        wid = sid * _NC + cid
        rbase = sid * nps
        pltpu.sync_copy(z_hbm.at[pl.ds(rbase, nps)], acc_sh.at[pl.ds(rbase, nps)])
        plsc.subcore_barrier()

        base = wid * epw

        @pl.loop(0, epw, step=c)
        def _(off):
            b = base + off
            pltpu.sync_copy(col_hbm.at[pl.ds(b, c)], idx_v)
            pltpu.sync_copy(msg_hbm.at[pl.ds(b, c)], buf_v)
            pltpu.sync_copy(buf_v, acc_sh.at[idx_v], add=True)

        plsc.subcore_barrier()
        pltpu.sync_copy(acc_sh.at[pl.ds(rbase, nps)],
                        out_hbm.at[pl.ds(cid * npad + rbase, nps)])

    out = k(msg, col, zeros)
    return out[:n], out[npad:npad + n]


def _tc_project(x, Pw):
    def body(x_ref, w_ref, o_ref):
        o_ref[...] = jnp.dot(x_ref[...], w_ref[...],
                             preferred_element_type=F32, precision=_HI)

    return pl.pallas_call(
        body,
        out_shape=jax.ShapeDtypeStruct((x.shape[0], Pw.shape[1]), F32),
    )(x, Pw)


_PACK = 4


def _bd(W, k):
    dk, dn = W.shape
    out = jnp.zeros((k * dk, k * dn), W.dtype)
    for i in range(k):
        out = out.at[i * dk:(i + 1) * dk, i * dn:(i + 1) * dn].set(W)
    return out


def _tc_edge(pr_, pc_, ea, w, block_e=3200):
    e = pr_.shape[0]
    de = ea.shape[1]
    dh = w["W3"].shape[1]
    k = _PACK
    ep = e // k
    bp = block_e // k
    d = pr_.shape[1]

    prp = pr_.reshape(ep, k * d)
    pcp = pc_.reshape(ep, k * d)
    eap = ea.reshape(ep, k * de)

    C1p = _bd(w["C1"], k)
    b1p = jnp.tile(w["b1"], (1, k))
    W2p = _bd(w["W2"], k)
    b2p = jnp.tile(w["b2"], (1, k))
    W3p = _bd(w["W3"], k)
    b3p = jnp.tile(w["b3"], (1, k))
    U1ap = _bd(w["V"][:de], k)
    Wvbp = _bd(w["V"][de:], k)
    vbp = jnp.tile(w["vb"], (1, k))
    U2p = _bd(w["U2"], k)
    ub2p = jnp.tile(w["ub2"], (1, k))
    U3p = _bd(w["U3"], k)
    ub3p = jnp.tile(w["ub3"], (1, k))

    def body(pr_ref, pc_ref, ea_ref, C1r, b1r, W2r, b2r, W3r, b3r,
             U1ar, Wvbr, vbr, U2r, ub2r, U3r, ub3r, msg_ref, ea2_ref):
        pr = pr_ref[...]
        pc = pc_ref[...]
        ea_ = ea_ref[...]
        dot = lambda a, b: jnp.dot(a, b, preferred_element_type=F32,
                                   precision=_HI)
        pra = jnp.concatenate([pr[:, i * d:i * d + 64] for i in range(k)], axis=1)
        prb = jnp.concatenate([pr[:, i * d + 64:(i + 1) * d] for i in range(k)], axis=1)
        pca = jnp.concatenate([pc[:, i * d:i * d + 64] for i in range(k)], axis=1)
        pcb = jnp.concatenate([pc[:, i * d + 64:(i + 1) * d] for i in range(k)], axis=1)
        h1 = jnp.maximum(pra + pcb + dot(ea_, C1r[...]) + b1r[...], 0.0)
        h2 = jnp.maximum(dot(h1, W2r[...]) + b2r[...], 0.0)
        t = jnp.maximum(dot(ea_, U1ar[...]) + dot(h2, Wvbr[...]) + vbr[...], 0.0)
        t = jnp.maximum(dot(t, U2r[...]) + ub2r[...], 0.0)
        ea2 = dot(t, U3r[...]) + ub3r[...]
        ea2_ref[...] = ea2
        g1 = jnp.maximum(pca + prb + dot(ea2, C1r[...]) + b1r[...], 0.0)
        g2 = jnp.maximum(dot(g1, W2r[...]) + b2r[...], 0.0)
        msg_ref[...] = dot(g2, W3r[...]) + b3r[...]

    full = lambda a: pl.BlockSpec(a.shape, lambda i: (0,) * a.ndim)
    weights = [C1p, b1p, W2p, b2p, W3p, b3p,
               U1ap, Wvbp, vbp, U2p, ub2p, U3p, ub3p]
    msg_p, ea2_p = pl.pallas_call(
        body,
        grid=(ep // bp,),
        in_specs=[
            pl.BlockSpec((bp, k * d), lambda i: (i, 0)),
            pl.BlockSpec((bp, k * d), lambda i: (i, 0)),
            pl.BlockSpec((bp, k * de), lambda i: (i, 0)),
        ] + [full(a) for a in weights],
        out_specs=[
            pl.BlockSpec((bp, k * dh), lambda i: (i, 0)),
            pl.BlockSpec((bp, k * de), lambda i: (i, 0)),
        ],
        out_shape=[
            jax.ShapeDtypeStruct((ep, k * dh), F32),
            jax.ShapeDtypeStruct((ep, k * de), F32),
        ],
    )(prp, pcp, eap, *weights)
    return msg_p.reshape(e, dh), ea2_p.reshape(e, de)


def _tc_node(x, p0, p1, nw, residual, Pw_next):
    n, d = x.shape
    N1a, N1b = nw[0][0][:d], nw[0][0][d:]
    nb1 = nw[0][1]
    N2, nb2 = nw[1]
    N3, nb3 = nw[2]
    project = Pw_next is not None

    def dotw(a, b):
        return jnp.dot(a, b, preferred_element_type=F32, precision=_HI)

    args = [x, p0, p1, N1a, N1b, nb1, N2, nb2, N3, nb3]
    out_shape = [jax.ShapeDtypeStruct((n, d), F32)]
    if project:
        args.append(Pw_next)
        out_shape.append(jax.ShapeDtypeStruct((n, Pw_next.shape[1]), F32))

    def body2(*refs):
        nrefs = len(args)
        in_refs, out_refs = refs[:nrefs], refs[nrefs:]
        if project:
            (x_ref, p0_ref, p1_ref, N1ar, N1br, nb1r, N2r, nb2r, N3r, nb3r,
             Pwr) = in_refs
        else:
            (x_ref, p0_ref, p1_ref, N1ar, N1br, nb1r, N2r, nb2r, N3r,
             nb3r) = in_refs
        agg = p0_ref[...] + p1_ref[...]
        xv = x_ref[...]
        h = jnp.maximum(dotw(xv, N1ar[...]) + dotw(agg, N1br[...]) + nb1r[...], 0.0)
        h = jnp.maximum(dotw(h, N2r[...]) + nb2r[...], 0.0)
        o = dotw(h, N3r[...]) + nb3r[...]
        if residual:
            o = o + xv
        out_refs[0][...] = o
        if project:
            out_refs[1][...] = dotw(o, Pwr[...])

    outs = pl.pallas_call(body2, out_shape=out_shape)(*args)
    return outs if project else (outs[0], None)


def kernel(x, edge_attr, edge_index, params):
    row = edge_index[0]
    col = edge_index[1]
    n = x.shape[0]
    ws = [_prep_layer(lp) for lp in params]
    proj = _tc_project(x, ws[0]["Pw"])
    for i, w in enumerate(ws):
        pr_, pc_ = _sc_gather2(proj, row, col)
        msg, edge_attr = _tc_edge(pr_, pc_, edge_attr, w)
        p0, p1 = _sc_segment_sum(msg, col, n)
        pw_next = ws[i + 1]["Pw"] if i + 1 < len(ws) else None
        x, proj = _tc_node(x, p0, p1, w["nw"], residual=(i > 0),
                           Pw_next=pw_next)
    return x, edge_attr


# speedup vs baseline: 2.0009x; 1.0272x over previous
import functools

import jax
import jax.numpy as jnp
from jax import lax
from jax.experimental import pallas as pl
from jax.experimental.pallas import tpu as pltpu
from jax.experimental.pallas import tpu_sc as plsc

F32 = jnp.float32
_NC, _NS = 2, 16
_NW = _NC * _NS
_CHUNK = 80
_HI = lax.Precision.HIGHEST
_MED = lax.Precision.HIGH


def _fold_mlp(layers, eps=1e-5):
    folded = []
    n = len(layers)
    for i, p in enumerate(layers):
        W, b = p["W"], p["b"]
        if i < n - 1:
            s = p["g"] * lax.rsqrt(p["v"] + eps)
            W = W * s[None, :]
            b = b * s + (p["be"] - p["m"] * s)
        folded.append((W, b.reshape(1, -1)))
    return folded


def _prep_layer(lp):
    ew = _fold_mlp(lp["edge_mlp"])
    uw = _fold_mlp(lp["edge_update_mlp"])
    nw = _fold_mlp(lp["node_mlp"])
    W1, b1 = ew[0]
    d = 128
    de = 16
    A1, B1, C1 = W1[:d], W1[d:2 * d], W1[2 * d:]
    W2, b2 = ew[1]
    W3, b3 = ew[2]
    U1, ub1 = uw[0]
    U1a, U1b = U1[:de], U1[de:]
    V = jnp.concatenate([U1a, W3 @ U1b], axis=0)
    vb = ub1 + b3 @ U1b
    U2, ub2 = uw[1]
    U3, ub3 = uw[2]
    Pw = jnp.concatenate([A1, B1], axis=1)
    return {
        "Pw": Pw, "C1": C1, "b1": b1, "W2": W2, "b2": b2, "W3": W3, "b3": b3,
        "V": V, "vb": vb, "U2": U2, "ub2": ub2, "U3": U3, "ub3": ub3,
        "nw": nw,
    }


def _sc_gather2(p, row, col):
    n, d = p.shape
    dt = p.dtype
    e = row.shape[0]
    epw = e // _NW
    c = _CHUNK
    mesh = plsc.VectorSubcoreMesh(core_axis_name="c", subcore_axis_name="s")

    @functools.partial(
        pl.kernel,
        mesh=mesh,
        out_type=(jax.ShapeDtypeStruct((e, d), dt),
                  jax.ShapeDtypeStruct((e, d), dt)),
        scratch_types=[
            pltpu.VMEM((c,), jnp.int32), pltpu.VMEM((c, d), dt),
            pltpu.VMEM((c,), jnp.int32), pltpu.VMEM((c, d), dt),
            pltpu.SemaphoreType.DMA, pltpu.SemaphoreType.DMA,
        ],
    )
    def k(p_hbm, row_hbm, col_hbm, outr_hbm, outc_hbm,
          idx_r, buf_r, idx_c, buf_c, sem_r, sem_c):
        wid = lax.axis_index("s") * _NC + lax.axis_index("c")
        base = wid * epw

        @pl.loop(0, epw, step=c)
        def _(off):
            b = base + off
            pltpu.sync_copy(row_hbm.at[pl.ds(b, c)], idx_r)
            pltpu.sync_copy(col_hbm.at[pl.ds(b, c)], idx_c)
            cp_r = pltpu.async_copy(p_hbm.at[idx_r], buf_r, sem_r)
            cp_c = pltpu.async_copy(p_hbm.at[idx_c], buf_c, sem_c)
            cp_r.wait()
            cp_c.wait()
            pltpu.sync_copy(buf_r, outr_hbm.at[pl.ds(b, c)])
            pltpu.sync_copy(buf_c, outc_hbm.at[pl.ds(b, c)])

    return k(p, row, col)


def _sc_segment_sum(msg, col, n):
    e, d = msg.shape
    epw = e // _NW
    c = _CHUNK
    npad = -(-n // 128) * 128
    nps = npad // _NS
    zeros = jnp.zeros((npad, d), F32)
    mesh = plsc.VectorSubcoreMesh(core_axis_name="c", subcore_axis_name="s")

    @functools.partial(
        pl.kernel,
        mesh=mesh,
        out_type=jax.ShapeDtypeStruct((_NC * npad, d), F32),
        scratch_types=[
            pltpu.VMEM((c,), jnp.int32), pltpu.VMEM((c, d), F32),
            pltpu.VMEM_SHARED((npad, d), F32),
            pltpu.SemaphoreType.DMA,
        ],
        compiler_params=pltpu.CompilerParams(use_tc_tiling_on_sc=False),
    )
    def k(msg_hbm, col_hbm, z_hbm, out_hbm, idx_v, buf_v, acc_sh, sem):
        cid = lax.axis_index("c")
        sid = lax.axis_index("s")
        wid = sid * _NC + cid
        rbase = sid * nps
        pltpu.sync_copy(z_hbm.at[pl.ds(rbase, nps)], acc_sh.at[pl.ds(rbase, nps)])
        plsc.subcore_barrier()

        base = wid * epw

        @pl.loop(0, epw, step=c)
        def _(off):
            b = base + off
            pltpu.sync_copy(col_hbm.at[pl.ds(b, c)], idx_v)
            pltpu.sync_copy(msg_hbm.at[pl.ds(b, c)], buf_v)
            pltpu.sync_copy(buf_v, acc_sh.at[idx_v], add=True)

        plsc.subcore_barrier()
        pltpu.sync_copy(acc_sh.at[pl.ds(rbase, nps)],
                        out_hbm.at[pl.ds(cid * npad + rbase, nps)])

    out = k(msg, col, zeros)
    return out[:n], out[npad:npad + n]


def _tc_project(x, Pw):
    def body(x_ref, w_ref, o_ref):
        o_ref[...] = jnp.dot(x_ref[...], w_ref[...],
                             preferred_element_type=F32, precision=_HI)

    return pl.pallas_call(
        body,
        out_shape=jax.ShapeDtypeStruct((x.shape[0], Pw.shape[1]), F32),
    )(x, Pw)


_PACK = 4


def _bd(W, k):
    dk, dn = W.shape
    out = jnp.zeros((k * dk, k * dn), W.dtype)
    for i in range(k):
        out = out.at[i * dk:(i + 1) * dk, i * dn:(i + 1) * dn].set(W)
    return out


def _tc_edge(pr_, pc_, ea, w, block_e=3200):
    e = pr_.shape[0]
    de = ea.shape[1]
    dh = w["W3"].shape[1]
    k = _PACK
    ep = e // k
    bp = block_e // k
    d = pr_.shape[1]

    prp = pr_.reshape(ep, k * d)
    pcp = pc_.reshape(ep, k * d)
    eap = ea.reshape(ep, k * de)

    C1p = _bd(w["C1"], k)
    b1p = jnp.tile(w["b1"], (1, k))
    W2p = _bd(w["W2"], k)
    b2p = jnp.tile(w["b2"], (1, k))
    W3p = _bd(w["W3"], k)
    b3p = jnp.tile(w["b3"], (1, k))
    U1ap = _bd(w["V"][:de], k)
    Wvbp = _bd(w["V"][de:], k)
    vbp = jnp.tile(w["vb"], (1, k))
    U2p = _bd(w["U2"], k)
    ub2p = jnp.tile(w["ub2"], (1, k))
    U3p = _bd(w["U3"], k)
    ub3p = jnp.tile(w["ub3"], (1, k))

    def body(pr_ref, pc_ref, ea_ref, C1r, b1r, W2r, b2r, W3r, b3r,
             U1ar, Wvbr, vbr, U2r, ub2r, U3r, ub3r, msg_ref, ea2_ref):
        pr = pr_ref[...]
        pc = pc_ref[...]
        ea_ = ea_ref[...]
        dot = lambda a, b: jnp.dot(a, b, preferred_element_type=F32,
                                   precision=_HI)
        cat = lambda xs: jnp.concatenate(xs, axis=1).astype(F32)
        pra = cat([pr[:, i * d:i * d + 64] for i in range(k)])
        prb = cat([pr[:, i * d + 64:(i + 1) * d] for i in range(k)])
        pca = cat([pc[:, i * d:i * d + 64] for i in range(k)])
        pcb = cat([pc[:, i * d + 64:(i + 1) * d] for i in range(k)])
        h1 = jnp.maximum(pra + pcb + dot(ea_, C1r[...]) + b1r[...], 0.0)
        h2 = jnp.maximum(dot(h1, W2r[...]) + b2r[...], 0.0)
        t = jnp.maximum(dot(ea_, U1ar[...]) + dot(h2, Wvbr[...]) + vbr[...], 0.0)
        t = jnp.maximum(dot(t, U2r[...]) + ub2r[...], 0.0)
        ea2 = dot(t, U3r[...]) + ub3r[...]
        ea2_ref[...] = ea2
        g1 = jnp.maximum(pca + prb + dot(ea2, C1r[...]) + b1r[...], 0.0)
        g2 = jnp.maximum(dot(g1, W2r[...]) + b2r[...], 0.0)
        msg_ref[...] = dot(g2, W3r[...]) + b3r[...]

    full = lambda a: pl.BlockSpec(a.shape, lambda i: (0,) * a.ndim)
    weights = [C1p, b1p, W2p, b2p, W3p, b3p,
               U1ap, Wvbp, vbp, U2p, ub2p, U3p, ub3p]
    msg_p, ea2_p = pl.pallas_call(
        body,
        grid=(ep // bp,),
        in_specs=[
            pl.BlockSpec((bp, k * d), lambda i: (i, 0)),
            pl.BlockSpec((bp, k * d), lambda i: (i, 0)),
            pl.BlockSpec((bp, k * de), lambda i: (i, 0)),
        ] + [full(a) for a in weights],
        out_specs=[
            pl.BlockSpec((bp, k * dh), lambda i: (i, 0)),
            pl.BlockSpec((bp, k * de), lambda i: (i, 0)),
        ],
        out_shape=[
            jax.ShapeDtypeStruct((ep, k * dh), F32),
            jax.ShapeDtypeStruct((ep, k * de), F32),
        ],
    )(prp, pcp, eap, *weights)
    return msg_p.reshape(e, dh), ea2_p.reshape(e, de)


def _tc_node(x, parts, nw, residual, Pw_next):
    n, d = x.shape
    np_ = len(parts)
    N1a, N1b = nw[0][0][:d], nw[0][0][d:]
    nb1 = nw[0][1]
    N2, nb2 = nw[1]
    N3, nb3 = nw[2]
    project = Pw_next is not None

    def dotw(a, b):
        return jnp.dot(a, b, preferred_element_type=F32, precision=_HI)

    args = [x, *parts, N1a, N1b, nb1, N2, nb2, N3, nb3]
    out_shape = [jax.ShapeDtypeStruct((n, d), F32)]
    if project:
        args.append(Pw_next)
        out_shape.append(jax.ShapeDtypeStruct((n, Pw_next.shape[1]), F32))

    def body2(*refs):
        nrefs = len(args)
        in_refs, out_refs = refs[:nrefs], refs[nrefs:]
        x_ref = in_refs[0]
        p_refs = in_refs[1:1 + np_]
        N1ar, N1br, nb1r, N2r, nb2r, N3r, nb3r = in_refs[1 + np_:8 + np_]
        agg = p_refs[0][...]
        for pr in p_refs[1:]:
            agg = agg + pr[...]
        xv = x_ref[...]
        h = jnp.maximum(dotw(xv, N1ar[...]) + dotw(agg, N1br[...]) + nb1r[...], 0.0)
        h = jnp.maximum(dotw(h, N2r[...]) + nb2r[...], 0.0)
        o = dotw(h, N3r[...]) + nb3r[...]
        if residual:
            o = o + xv
        out_refs[0][...] = o
        if project:
            out_refs[1][...] = dotw(o, in_refs[8 + np_][...])

    outs = pl.pallas_call(body2, out_shape=out_shape)(*args)
    return outs if project else (outs[0], None)


def kernel(x, edge_attr, edge_index, params):
    row = edge_index[0]
    col = edge_index[1]
    n = x.shape[0]
    e = row.shape[0]
    he = (e * 16 // 31) // (_NW * _CHUNK * _PACK) * (_NW * _CHUNK * _PACK)
    halves = ((row[:he], col[:he]), (row[he:], col[he:]))
    eas = [edge_attr[:he], edge_attr[he:]]
    ws = [_prep_layer(lp) for lp in params]
    proj = _tc_project(x, ws[0]["Pw"])
    for i, w in enumerate(ws):
        parts = []
        msgs = [None, None]
        for h in (0, 1):
            r_, c_ = halves[h]
            pr_, pc_ = _sc_gather2(proj, r_, c_)
            msgs[h], eas[h] = _tc_edge(pr_, pc_, eas[h], w, block_e=2560)
            parts.extend(_sc_segment_sum(msgs[h], c_, n))
        pw_next = ws[i + 1]["Pw"] if i + 1 < len(ws) else None
        x, proj = _tc_node(x, parts, w["nw"], residual=(i > 0),
                           Pw_next=pw_next)
    return x, jnp.concatenate(eas, axis=0)


# speedup vs baseline: 2.0634x; 1.0312x over previous
import functools

import jax
import jax.numpy as jnp
from jax import lax
from jax.experimental import pallas as pl
from jax.experimental.pallas import tpu as pltpu
from jax.experimental.pallas import tpu_sc as plsc

F32 = jnp.float32
_NC, _NS = 2, 16
_NW = _NC * _NS
_CHUNK = 80
_HI = lax.Precision.HIGHEST
_MED = lax.Precision.HIGH


def _fold_mlp(layers, eps=1e-5):
    folded = []
    n = len(layers)
    for i, p in enumerate(layers):
        W, b = p["W"], p["b"]
        if i < n - 1:
            s = p["g"] * lax.rsqrt(p["v"] + eps)
            W = W * s[None, :]
            b = b * s + (p["be"] - p["m"] * s)
        folded.append((W, b.reshape(1, -1)))
    return folded


def _prep_layer(lp):
    ew = _fold_mlp(lp["edge_mlp"])
    uw = _fold_mlp(lp["edge_update_mlp"])
    nw = _fold_mlp(lp["node_mlp"])
    W1, b1 = ew[0]
    d = 128
    de = 16
    A1, B1, C1 = W1[:d], W1[d:2 * d], W1[2 * d:]
    W2, b2 = ew[1]
    W3, b3 = ew[2]
    U1, ub1 = uw[0]
    U1a, U1b = U1[:de], U1[de:]
    V = jnp.concatenate([U1a, W3 @ U1b], axis=0)
    vb = ub1 + b3 @ U1b
    U2, ub2 = uw[1]
    U3, ub3 = uw[2]
    Pw = jnp.concatenate([A1, B1], axis=1)
    return {
        "Pw": Pw, "C1": C1, "b1": b1, "W2": W2, "b2": b2, "W3": W3, "b3": b3,
        "V": V, "vb": vb, "U2": U2, "ub2": ub2, "U3": U3, "ub3": ub3,
        "nw": nw,
    }


def _sc_gather2(p, row, col):
    n, d = p.shape
    dt = p.dtype
    e = row.shape[0]
    epw = e // _NW
    c = _CHUNK
    mesh = plsc.VectorSubcoreMesh(core_axis_name="c", subcore_axis_name="s")

    @functools.partial(
        pl.kernel,
        mesh=mesh,
        out_type=(jax.ShapeDtypeStruct((e, d), dt),
                  jax.ShapeDtypeStruct((e, d), dt)),
        scratch_types=[
            pltpu.VMEM((c,), jnp.int32), pltpu.VMEM((c, d), dt),
            pltpu.VMEM((c,), jnp.int32), pltpu.VMEM((c, d), dt),
            pltpu.SemaphoreType.DMA, pltpu.SemaphoreType.DMA,
            pltpu.VMEM((c,), jnp.int32), pltpu.VMEM((c, d), dt),
            pltpu.VMEM((c,), jnp.int32), pltpu.VMEM((c, d), dt),
            pltpu.SemaphoreType.DMA, pltpu.SemaphoreType.DMA,
        ],
    )
    def k(p_hbm, row_hbm, col_hbm, outr_hbm, outc_hbm,
          idx_r0, buf_r0, idx_c0, buf_c0, sem_r0, sem_c0,
          idx_r1, buf_r1, idx_c1, buf_c1, sem_r1, sem_c1):
        wid = lax.axis_index("s") * _NC + lax.axis_index("c")
        base = wid * epw
        nch = epw // c
        banks = ((idx_r0, buf_r0, idx_c0, buf_c0, sem_r0, sem_c0),
                 (idx_r1, buf_r1, idx_c1, buf_c1, sem_r1, sem_c1))

        def load_idx(j, bank):
            ir, _, ic, _, _, _ = banks[bank]
            b = base + j * c
            pltpu.sync_copy(row_hbm.at[pl.ds(b, c)], ir)
            pltpu.sync_copy(col_hbm.at[pl.ds(b, c)], ic)

        def start_gather(bank):
            ir, br, ic, bc, sr, sc_ = banks[bank]
            pltpu.make_async_copy(p_hbm.at[ir], br, sr).start()
            pltpu.make_async_copy(p_hbm.at[ic], bc, sc_).start()

        def wait_gather(bank):
            ir, br, ic, bc, sr, sc_ = banks[bank]
            pltpu.make_async_copy(p_hbm.at[ir], br, sr).wait()
            pltpu.make_async_copy(p_hbm.at[ic], bc, sc_).wait()

        def write_out(j, bank):
            _, br, _, bc, _, _ = banks[bank]
            b = base + j * c
            pltpu.sync_copy(br, outr_hbm.at[pl.ds(b, c)])
            pltpu.sync_copy(bc, outc_hbm.at[pl.ds(b, c)])

        load_idx(0, 0)
        start_gather(0)

        @pl.loop(0, nch, step=2)
        def _(j):
            @pl.when(j + 1 < nch)
            def _():
                load_idx(j + 1, 1)
            wait_gather(0)

            @pl.when(j + 1 < nch)
            def _():
                start_gather(1)
            write_out(j, 0)

            @pl.when(j + 2 < nch)
            def _():
                load_idx(j + 2, 0)
                start_gather(0)

            @pl.when(j + 1 < nch)
            def _():
                wait_gather(1)
                write_out(j + 1, 1)

    return k(p, row, col)


def _sc_segment_sum(msg, col, n):
    e, d = msg.shape
    epw = e // _NW
    c = _CHUNK
    npad = -(-n // 128) * 128
    nps = npad // _NS
    zeros = jnp.zeros((npad, d), F32)
    mesh = plsc.VectorSubcoreMesh(core_axis_name="c", subcore_axis_name="s")

    @functools.partial(
        pl.kernel,
        mesh=mesh,
        out_type=jax.ShapeDtypeStruct((_NC * npad, d), F32),
        scratch_types=[
            pltpu.VMEM((c,), jnp.int32), pltpu.VMEM((c, d), F32),
            pltpu.SemaphoreType.DMA,
            pltpu.VMEM((c,), jnp.int32), pltpu.VMEM((c, d), F32),
            pltpu.SemaphoreType.DMA,
            pltpu.VMEM_SHARED((npad, d), F32),
        ],
        compiler_params=pltpu.CompilerParams(use_tc_tiling_on_sc=False),
    )
    def k(msg_hbm, col_hbm, z_hbm, out_hbm,
          idx0, buf0, sem0, idx1, buf1, sem1, acc_sh):
        cid = lax.axis_index("c")
        sid = lax.axis_index("s")
        wid = sid * _NC + cid
        rbase = sid * nps
        pltpu.sync_copy(z_hbm.at[pl.ds(rbase, nps)], acc_sh.at[pl.ds(rbase, nps)])
        plsc.subcore_barrier()

        base = wid * epw
        nch = epw // c
        banks = ((idx0, buf0, sem0), (idx1, buf1, sem1))

        def load(j, bank):
            ix, bf, _ = banks[bank]
            b = base + j * c
            pltpu.sync_copy(col_hbm.at[pl.ds(b, c)], ix)
            pltpu.sync_copy(msg_hbm.at[pl.ds(b, c)], bf)

        def start_scatter(bank):
            ix, bf, sm = banks[bank]
            pltpu.async_copy(bf, acc_sh.at[ix], sm, add=True)

        def wait_scatter(bank):
            ix, bf, sm = banks[bank]
            pltpu.make_async_copy(bf, acc_sh.at[ix], sm).wait()

        load(0, 0)

        @pl.loop(0, nch, step=2)
        def _(j):
            start_scatter(0)

            @pl.when(j + 1 < nch)
            def _():
                load(j + 1, 1)
                start_scatter(1)
            wait_scatter(0)

            @pl.when(j + 2 < nch)
            def _():
                load(j + 2, 0)

            @pl.when(j + 1 < nch)
            def _():
                wait_scatter(1)

        plsc.subcore_barrier()
        pltpu.sync_copy(acc_sh.at[pl.ds(rbase, nps)],
                        out_hbm.at[pl.ds(cid * npad + rbase, nps)])

    out = k(msg, col, zeros)
    return out[:n], out[npad:npad + n]


def _tc_project(x, Pw):
    def body(x_ref, w_ref, o_ref):
        o_ref[...] = jnp.dot(x_ref[...], w_ref[...],
                             preferred_element_type=F32, precision=_HI)

    return pl.pallas_call(
        body,
        out_shape=jax.ShapeDtypeStruct((x.shape[0], Pw.shape[1]), F32),
    )(x, Pw)


_PACK = 4


def _bd(W, k):
    dk, dn = W.shape
    out = jnp.zeros((k * dk, k * dn), W.dtype)
    for i in range(k):
        out = out.at[i * dk:(i + 1) * dk, i * dn:(i + 1) * dn].set(W)
    return out


def _tc_edge(pr_, pc_, ea, w, block_e=3200):
    e = pr_.shape[0]
    de = ea.shape[1]
    dh = w["W3"].shape[1]
    k = _PACK
    ep = e // k
    bp = block_e // k
    d = pr_.shape[1]

    prp = pr_.reshape(ep, k * d)
    pcp = pc_.reshape(ep, k * d)
    eap = ea.reshape(ep, k * de)

    C1p = _bd(w["C1"], k)
    b1p = jnp.tile(w["b1"], (1, k))
    W2p = _bd(w["W2"], k)
    b2p = jnp.tile(w["b2"], (1, k))
    W3p = _bd(w["W3"], k)
    b3p = jnp.tile(w["b3"], (1, k))
    U1ap = _bd(w["V"][:de], k)
    Wvbp = _bd(w["V"][de:], k)
    vbp = jnp.tile(w["vb"], (1, k))
    U2p = _bd(w["U2"], k)
    ub2p = jnp.tile(w["ub2"], (1, k))
    U3p = _bd(w["U3"], k)
    ub3p = jnp.tile(w["ub3"], (1, k))

    def body(pr_ref, pc_ref, ea_ref, C1r, b1r, W2r, b2r, W3r, b3r,
             U1ar, Wvbr, vbr, U2r, ub2r, U3r, ub3r, msg_ref, ea2_ref):
        pr = pr_ref[...]
        pc = pc_ref[...]
        ea_ = ea_ref[...]
        dot = lambda a, b: jnp.dot(a, b, preferred_element_type=F32,
                                   precision=_HI)
        cat = lambda xs: jnp.concatenate(xs, axis=1).astype(F32)
        pra = cat([pr[:, i * d:i * d + 64] for i in range(k)])
        prb = cat([pr[:, i * d + 64:(i + 1) * d] for i in range(k)])
        pca = cat([pc[:, i * d:i * d + 64] for i in range(k)])
        pcb = cat([pc[:, i * d + 64:(i + 1) * d] for i in range(k)])
        h1 = jnp.maximum(pra + pcb + dot(ea_, C1r[...]) + b1r[...], 0.0)
        h2 = jnp.maximum(dot(h1, W2r[...]) + b2r[...], 0.0)
        t = jnp.maximum(dot(ea_, U1ar[...]) + dot(h2, Wvbr[...]) + vbr[...], 0.0)
        t = jnp.maximum(dot(t, U2r[...]) + ub2r[...], 0.0)
        ea2 = dot(t, U3r[...]) + ub3r[...]
        ea2_ref[...] = ea2
        g1 = jnp.maximum(pca + prb + dot(ea2, C1r[...]) + b1r[...], 0.0)
        g2 = jnp.maximum(dot(g1, W2r[...]) + b2r[...], 0.0)
        msg_ref[...] = dot(g2, W3r[...]) + b3r[...]

    full = lambda a: pl.BlockSpec(a.shape, lambda i: (0,) * a.ndim)
    weights = [C1p, b1p, W2p, b2p, W3p, b3p,
               U1ap, Wvbp, vbp, U2p, ub2p, U3p, ub3p]
    msg_p, ea2_p = pl.pallas_call(
        body,
        grid=(ep // bp,),
        in_specs=[
            pl.BlockSpec((bp, k * d), lambda i: (i, 0)),
            pl.BlockSpec((bp, k * d), lambda i: (i, 0)),
            pl.BlockSpec((bp, k * de), lambda i: (i, 0)),
        ] + [full(a) for a in weights],
        out_specs=[
            pl.BlockSpec((bp, k * dh), lambda i: (i, 0)),
            pl.BlockSpec((bp, k * de), lambda i: (i, 0)),
        ],
        out_shape=[
            jax.ShapeDtypeStruct((ep, k * dh), F32),
            jax.ShapeDtypeStruct((ep, k * de), F32),
        ],
    )(prp, pcp, eap, *weights)
    return msg_p.reshape(e, dh), ea2_p.reshape(e, de)


def _tc_node(x, parts, nw, residual, Pw_next):
    n, d = x.shape
    np_ = len(parts)
    N1a, N1b = nw[0][0][:d], nw[0][0][d:]
    nb1 = nw[0][1]
    N2, nb2 = nw[1]
    N3, nb3 = nw[2]
    project = Pw_next is not None

    def dotw(a, b):
        return jnp.dot(a, b, preferred_element_type=F32, precision=_HI)

    args = [x, *parts, N1a, N1b, nb1, N2, nb2, N3, nb3]
    out_shape = [jax.ShapeDtypeStruct((n, d), F32)]
    if project:
        args.append(Pw_next)
        out_shape.append(jax.ShapeDtypeStruct((n, Pw_next.shape[1]), F32))

    def body2(*refs):
        nrefs = len(args)
        in_refs, out_refs = refs[:nrefs], refs[nrefs:]
        x_ref = in_refs[0]
        p_refs = in_refs[1:1 + np_]
        N1ar, N1br, nb1r, N2r, nb2r, N3r, nb3r = in_refs[1 + np_:8 + np_]
        agg = p_refs[0][...]
        for pr in p_refs[1:]:
            agg = agg + pr[...]
        xv = x_ref[...]
        h = jnp.maximum(dotw(xv, N1ar[...]) + dotw(agg, N1br[...]) + nb1r[...], 0.0)
        h = jnp.maximum(dotw(h, N2r[...]) + nb2r[...], 0.0)
        o = dotw(h, N3r[...]) + nb3r[...]
        if residual:
            o = o + xv
        out_refs[0][...] = o
        if project:
            out_refs[1][...] = dotw(o, in_refs[8 + np_][...])

    outs = pl.pallas_call(body2, out_shape=out_shape)(*args)
    return outs if project else (outs[0], None)


def kernel(x, edge_attr, edge_index, params):
    row = edge_index[0]
    col = edge_index[1]
    n = x.shape[0]
    e = row.shape[0]
    he = (e * 16 // 31) // (_NW * _CHUNK * _PACK) * (_NW * _CHUNK * _PACK)
    halves = ((row[:he], col[:he]), (row[he:], col[he:]))
    eas = [edge_attr[:he], edge_attr[he:]]
    ws = [_prep_layer(lp) for lp in params]
    proj = _tc_project(x, ws[0]["Pw"])
    for i, w in enumerate(ws):
        parts = []
        msgs = [None, None]
        for h in (0, 1):
            r_, c_ = halves[h]
            pr_, pc_ = _sc_gather2(proj, r_, c_)
            msgs[h], eas[h] = _tc_edge(pr_, pc_, eas[h], w, block_e=2560)
            parts.extend(_sc_segment_sum(msgs[h], c_, n))
        pw_next = ws[i + 1]["Pw"] if i + 1 < len(ws) else None
        x, proj = _tc_node(x, parts, w["nw"], residual=(i > 0),
                           Pw_next=pw_next)
    return x, jnp.concatenate(eas, axis=0)


# speedup vs baseline: 2.1459x; 1.0400x over previous
import functools

import jax
import jax.numpy as jnp
from jax import lax
from jax.experimental import pallas as pl
from jax.experimental.pallas import tpu as pltpu
from jax.experimental.pallas import tpu_sc as plsc

F32 = jnp.float32
_NC, _NS = 2, 16
_NW = _NC * _NS
_CHUNK = 80
_HI = lax.Precision.HIGHEST
_MED = lax.Precision.HIGH


def _fold_mlp(layers, eps=1e-5):
    folded = []
    n = len(layers)
    for i, p in enumerate(layers):
        W, b = p["W"], p["b"]
        if i < n - 1:
            s = p["g"] * lax.rsqrt(p["v"] + eps)
            W = W * s[None, :]
            b = b * s + (p["be"] - p["m"] * s)
        folded.append((W, b.reshape(1, -1)))
    return folded


def _prep_layer(lp):
    ew = _fold_mlp(lp["edge_mlp"])
    uw = _fold_mlp(lp["edge_update_mlp"])
    nw = _fold_mlp(lp["node_mlp"])
    W1, b1 = ew[0]
    d = 128
    de = 16
    A1, B1, C1 = W1[:d], W1[d:2 * d], W1[2 * d:]
    W2, b2 = ew[1]
    W3, b3 = ew[2]
    U1, ub1 = uw[0]
    U1a, U1b = U1[:de], U1[de:]
    V = jnp.concatenate([U1a, W3 @ U1b], axis=0)
    vb = ub1 + b3 @ U1b
    U2, ub2 = uw[1]
    U3, ub3 = uw[2]
    Pw = jnp.concatenate([A1, B1], axis=1)
    return {
        "Pw": Pw, "C1": C1, "b1": b1, "W2": W2, "b2": b2, "W3": W3, "b3": b3,
        "V": V, "vb": vb, "U2": U2, "ub2": ub2, "U3": U3, "ub3": ub3,
        "nw": nw,
    }


def _sc_gather2(p, row, col):
    n, d = p.shape
    dt = p.dtype
    e = row.shape[0]
    epw = e // _NW
    c = _CHUNK
    mesh = plsc.VectorSubcoreMesh(core_axis_name="c", subcore_axis_name="s")

    @functools.partial(
        pl.kernel,
        mesh=mesh,
        out_type=(jax.ShapeDtypeStruct((e, d), dt),
                  jax.ShapeDtypeStruct((e, d), dt)),
        scratch_types=[
            pltpu.VMEM((c,), jnp.int32), pltpu.VMEM((c, d), dt),
            pltpu.VMEM((c,), jnp.int32), pltpu.VMEM((c, d), dt),
            pltpu.SemaphoreType.DMA, pltpu.SemaphoreType.DMA,
            pltpu.VMEM((c,), jnp.int32), pltpu.VMEM((c, d), dt),
            pltpu.VMEM((c,), jnp.int32), pltpu.VMEM((c, d), dt),
            pltpu.SemaphoreType.DMA, pltpu.SemaphoreType.DMA,
        ],
    )
    def k(p_hbm, row_hbm, col_hbm, outr_hbm, outc_hbm,
          idx_r0, buf_r0, idx_c0, buf_c0, sem_r0, sem_c0,
          idx_r1, buf_r1, idx_c1, buf_c1, sem_r1, sem_c1):
        wid = lax.axis_index("s") * _NC + lax.axis_index("c")
        base = wid * epw
        nch = epw // c
        banks = ((idx_r0, buf_r0, idx_c0, buf_c0, sem_r0, sem_c0),
                 (idx_r1, buf_r1, idx_c1, buf_c1, sem_r1, sem_c1))

        def load_idx(j, bank):
            ir, _, ic, _, _, _ = banks[bank]
            b = base + j * c
            pltpu.sync_copy(row_hbm.at[pl.ds(b, c)], ir)
            pltpu.sync_copy(col_hbm.at[pl.ds(b, c)], ic)

        def start_gather(bank):
            ir, br, ic, bc, sr, sc_ = banks[bank]
            pltpu.make_async_copy(p_hbm.at[ir], br, sr).start()
            pltpu.make_async_copy(p_hbm.at[ic], bc, sc_).start()

        def wait_gather(bank):
            ir, br, ic, bc, sr, sc_ = banks[bank]
            pltpu.make_async_copy(p_hbm.at[ir], br, sr).wait()
            pltpu.make_async_copy(p_hbm.at[ic], bc, sc_).wait()

        def write_out(j, bank):
            _, br, _, bc, _, _ = banks[bank]
            b = base + j * c
            pltpu.sync_copy(br, outr_hbm.at[pl.ds(b, c)])
            pltpu.sync_copy(bc, outc_hbm.at[pl.ds(b, c)])

        load_idx(0, 0)
        start_gather(0)

        @pl.loop(0, nch, step=2)
        def _(j):
            @pl.when(j + 1 < nch)
            def _():
                load_idx(j + 1, 1)
            wait_gather(0)

            @pl.when(j + 1 < nch)
            def _():
                start_gather(1)
            write_out(j, 0)

            @pl.when(j + 2 < nch)
            def _():
                load_idx(j + 2, 0)
                start_gather(0)

            @pl.when(j + 1 < nch)
            def _():
                wait_gather(1)
                write_out(j + 1, 1)

    return k(p, row, col)


def _sc_segment_sum(msg, col, n):
    e, d = msg.shape
    epw = e // _NW
    c = _CHUNK
    npad = -(-n // 128) * 128
    nps = npad // _NS
    zeros = jnp.zeros((npad, d), F32)
    mesh = plsc.VectorSubcoreMesh(core_axis_name="c", subcore_axis_name="s")

    @functools.partial(
        pl.kernel,
        mesh=mesh,
        out_type=jax.ShapeDtypeStruct((_NC * npad, d), F32),
        scratch_types=[
            pltpu.VMEM((c,), jnp.int32), pltpu.VMEM((c, d), F32),
            pltpu.SemaphoreType.DMA,
            pltpu.VMEM((c,), jnp.int32), pltpu.VMEM((c, d), F32),
            pltpu.SemaphoreType.DMA,
            pltpu.VMEM_SHARED((npad, d), F32),
        ],
        compiler_params=pltpu.CompilerParams(use_tc_tiling_on_sc=False),
    )
    def k(msg_hbm, col_hbm, z_hbm, out_hbm,
          idx0, buf0, sem0, idx1, buf1, sem1, acc_sh):
        cid = lax.axis_index("c")
        sid = lax.axis_index("s")
        wid = sid * _NC + cid
        rbase = sid * nps
        pltpu.sync_copy(z_hbm.at[pl.ds(rbase, nps)], acc_sh.at[pl.ds(rbase, nps)])
        plsc.subcore_barrier()

        base = wid * epw
        nch = epw // c
        banks = ((idx0, buf0, sem0), (idx1, buf1, sem1))

        def load(j, bank):
            ix, bf, _ = banks[bank]
            b = base + j * c
            pltpu.sync_copy(col_hbm.at[pl.ds(b, c)], ix)
            pltpu.sync_copy(msg_hbm.at[pl.ds(b, c)], bf)

        def start_scatter(bank):
            ix, bf, sm = banks[bank]
            pltpu.async_copy(bf, acc_sh.at[ix], sm, add=True)

        def wait_scatter(bank):
            ix, bf, sm = banks[bank]
            pltpu.make_async_copy(bf, acc_sh.at[ix], sm).wait()

        load(0, 0)

        @pl.loop(0, nch, step=2)
        def _(j):
            start_scatter(0)

            @pl.when(j + 1 < nch)
            def _():
                load(j + 1, 1)
                start_scatter(1)
            wait_scatter(0)

            @pl.when(j + 2 < nch)
            def _():
                load(j + 2, 0)

            @pl.when(j + 1 < nch)
            def _():
                wait_scatter(1)

        plsc.subcore_barrier()
        pltpu.sync_copy(acc_sh.at[pl.ds(rbase, nps)],
                        out_hbm.at[pl.ds(cid * npad + rbase, nps)])

    out = k(msg, col, zeros)
    return out[:n], out[npad:npad + n]


def _tc_project(x, Pw):
    def body(x_ref, w_ref, o_ref):
        o_ref[...] = jnp.dot(x_ref[...], w_ref[...],
                             preferred_element_type=F32, precision=_HI)

    return pl.pallas_call(
        body,
        out_shape=jax.ShapeDtypeStruct((x.shape[0], Pw.shape[1]), F32),
    )(x, Pw)


_PACK = 8


def _bd(W, k):
    dk, dn = W.shape
    out = jnp.zeros((k * dk, k * dn), W.dtype)
    for i in range(k):
        out = out.at[i * dk:(i + 1) * dk, i * dn:(i + 1) * dn].set(W)
    return out


def _tc_edge(pr_, pc_, ea, w, block_e=3200):
    e = pr_.shape[0]
    de = ea.shape[1]
    dh = w["W3"].shape[1]
    k = _PACK
    ep = e // k
    bp = block_e // k
    d = pr_.shape[1]

    prp = pr_.reshape(ep, k * d)
    pcp = pc_.reshape(ep, k * d)
    eap = ea.reshape(ep, k * de)

    C1p = _bd(w["C1"], k)
    b1p = jnp.tile(w["b1"], (1, k))
    W2p = _bd(w["W2"], k)
    b2p = jnp.tile(w["b2"], (1, k))
    W3p = _bd(w["W3"], k)
    b3p = jnp.tile(w["b3"], (1, k))
    U1ap = _bd(w["V"][:de], k)
    Wvbp = _bd(w["V"][de:], k)
    vbp = jnp.tile(w["vb"], (1, k))
    U2p = _bd(w["U2"], k)
    ub2p = jnp.tile(w["ub2"], (1, k))
    U3p = _bd(w["U3"], k)
    ub3p = jnp.tile(w["ub3"], (1, k))

    def body(pr_ref, pc_ref, ea_ref, C1r, b1r, W2r, b2r, W3r, b3r,
             U1ar, Wvbr, vbr, U2r, ub2r, U3r, ub3r, msg_ref, ea2_ref):
        pr = pr_ref[...]
        pc = pc_ref[...]
        ea_ = ea_ref[...]
        dot = lambda a, b: jnp.dot(a, b, preferred_element_type=F32,
                                   precision=_HI)
        cat = lambda xs: jnp.concatenate(xs, axis=1).astype(F32)
        pra = cat([pr[:, i * d:i * d + 64] for i in range(k)])
        prb = cat([pr[:, i * d + 64:(i + 1) * d] for i in range(k)])
        pca = cat([pc[:, i * d:i * d + 64] for i in range(k)])
        pcb = cat([pc[:, i * d + 64:(i + 1) * d] for i in range(k)])
        h1 = jnp.maximum(pra + pcb + dot(ea_, C1r[...]) + b1r[...], 0.0)
        h2 = jnp.maximum(dot(h1, W2r[...]) + b2r[...], 0.0)
        t = jnp.maximum(dot(ea_, U1ar[...]) + dot(h2, Wvbr[...]) + vbr[...], 0.0)
        t = jnp.maximum(dot(t, U2r[...]) + ub2r[...], 0.0)
        ea2 = dot(t, U3r[...]) + ub3r[...]
        ea2_ref[...] = ea2
        g1 = jnp.maximum(pca + prb + dot(ea2, C1r[...]) + b1r[...], 0.0)
        g2 = jnp.maximum(dot(g1, W2r[...]) + b2r[...], 0.0)
        msg_ref[...] = dot(g2, W3r[...]) + b3r[...]

    full = lambda a: pl.BlockSpec(a.shape, lambda i: (0,) * a.ndim)
    weights = [C1p, b1p, W2p, b2p, W3p, b3p,
               U1ap, Wvbp, vbp, U2p, ub2p, U3p, ub3p]
    msg_p, ea2_p = pl.pallas_call(
        body,
        grid=(ep // bp,),
        in_specs=[
            pl.BlockSpec((bp, k * d), lambda i: (i, 0)),
            pl.BlockSpec((bp, k * d), lambda i: (i, 0)),
            pl.BlockSpec((bp, k * de), lambda i: (i, 0)),
        ] + [full(a) for a in weights],
        out_specs=[
            pl.BlockSpec((bp, k * dh), lambda i: (i, 0)),
            pl.BlockSpec((bp, k * de), lambda i: (i, 0)),
        ],
        out_shape=[
            jax.ShapeDtypeStruct((ep, k * dh), F32),
            jax.ShapeDtypeStruct((ep, k * de), F32),
        ],
    )(prp, pcp, eap, *weights)
    return msg_p.reshape(e, dh), ea2_p.reshape(e, de)


def _tc_node(x, parts, nw, residual, Pw_next):
    n, d = x.shape
    np_ = len(parts)
    N1a, N1b = nw[0][0][:d], nw[0][0][d:]
    nb1 = nw[0][1]
    N2, nb2 = nw[1]
    N3, nb3 = nw[2]
    project = Pw_next is not None

    def dotw(a, b):
        return jnp.dot(a, b, preferred_element_type=F32, precision=_HI)

    args = [x, *parts, N1a, N1b, nb1, N2, nb2, N3, nb3]
    out_shape = [jax.ShapeDtypeStruct((n, d), F32)]
    if project:
        args.append(Pw_next)
        out_shape.append(jax.ShapeDtypeStruct((n, Pw_next.shape[1]), F32))

    def body2(*refs):
        nrefs = len(args)
        in_refs, out_refs = refs[:nrefs], refs[nrefs:]
        x_ref = in_refs[0]
        p_refs = in_refs[1:1 + np_]
        N1ar, N1br, nb1r, N2r, nb2r, N3r, nb3r = in_refs[1 + np_:8 + np_]
        agg = p_refs[0][...]
        for pr in p_refs[1:]:
            agg = agg + pr[...]
        xv = x_ref[...]
        h = jnp.maximum(dotw(xv, N1ar[...]) + dotw(agg, N1br[...]) + nb1r[...], 0.0)
        h = jnp.maximum(dotw(h, N2r[...]) + nb2r[...], 0.0)
        o = dotw(h, N3r[...]) + nb3r[...]
        if residual:
            o = o + xv
        out_refs[0][...] = o
        if project:
            out_refs[1][...] = dotw(o, in_refs[8 + np_][...])

    outs = pl.pallas_call(body2, out_shape=out_shape)(*args)
    return outs if project else (outs[0], None)


def kernel(x, edge_attr, edge_index, params):
    row = edge_index[0]
    col = edge_index[1]
    n = x.shape[0]
    e = row.shape[0]
    he = (e * 16 // 31) // (_NW * _CHUNK * _PACK) * (_NW * _CHUNK * _PACK)
    halves = ((row[:he], col[:he]), (row[he:], col[he:]))
    eas = [edge_attr[:he], edge_attr[he:]]
    ws = [_prep_layer(lp) for lp in params]
    proj = _tc_project(x, ws[0]["Pw"])
    for i, w in enumerate(ws):
        parts = []
        msgs = [None, None]
        for h in (0, 1):
            r_, c_ = halves[h]
            pr_, pc_ = _sc_gather2(proj, r_, c_)
            msgs[h], eas[h] = _tc_edge(pr_, pc_, eas[h], w, block_e=2560)
            parts.extend(_sc_segment_sum(msgs[h], c_, n))
        pw_next = ws[i + 1]["Pw"] if i + 1 < len(ws) else None
        x, proj = _tc_node(x, parts, w["nw"], residual=(i > 0),
                           Pw_next=pw_next)
    return x, jnp.concatenate(eas, axis=0)
